# Initial kernel scaffold; baseline (speedup 1.0000x reference)
import jax
import jax.numpy as jnp
from jax.experimental import pallas as pl


def kernel(x, edge_index, batch, W_conv, b_conv, W1, b1, W2, b2):
    raise NotImplementedError("write your pallas kernel here")


# speedup vs baseline: 3.8240x; 3.8240x over previous
import functools

import jax
import jax.numpy as jnp
from jax.experimental import pallas as pl
from jax.experimental.pallas import tpu as pltpu


def _finish_body(acc_ref, g_ref, dinv_ref, batch_ref, b_conv_ref,
                 W1_ref, b1_ref, W2_ref, b2_ref, out_ref):
    n = acc_ref.shape[0]
    n_graphs = 64
    dinv = dinv_ref[...]
    out_node = jnp.maximum((acc_ref[...] + g_ref[...]) * dinv + b_conv_ref[...], 0.0)
    gids = jax.lax.broadcasted_iota(jnp.int32, (n, n_graphs), 1)
    P = jnp.where(batch_ref[...] == gids, 1.0, 0.0)
    sums = jax.lax.dot_general(P, out_node, (((0,), (0,)), ((), ())),
                               preferred_element_type=jnp.float32)
    cnts = jnp.sum(P, axis=0, keepdims=True)
    pooled = sums / jnp.maximum(cnts, 1.0).T
    z = jnp.maximum(
        jax.lax.dot_general(pooled, W1_ref[...], (((1,), (0,)), ((), ())),
                            preferred_element_type=jnp.float32) + b1_ref[...], 0.0)
    out_ref[...] = jax.lax.dot_general(z, W2_ref[...], (((1,), (0,)), ((), ())),
                                       preferred_element_type=jnp.float32) + b2_ref[...]


def _finish(acc, g, dinv, batch, b_conv, W1, b1, W2, b2):
    n = acc.shape[0]
    return pl.pallas_call(
        _finish_body,
        out_shape=jax.ShapeDtypeStruct((64, 10), jnp.float32),
    )(acc, g, dinv, batch.reshape(n, 1), b_conv.reshape(1, -1),
      W1, b1.reshape(1, -1), W2, b2.reshape(1, -1))


def kernel(x, edge_index, batch, W_conv, b_conv, W1, b1, W2, b2):
    n = x.shape[0]
    src = edge_index[0]
    dst = edge_index[1]
    deg = jax.ops.segment_sum(jnp.ones_like(dst, dtype=x.dtype), dst,
                              num_segments=n) + 1.0
    dinv = jax.lax.rsqrt(deg)
    h = x @ W_conv
    g = h * dinv[:, None]
    acc = jax.ops.segment_sum(g[src], dst, num_segments=n)
    return _finish(acc, g, dinv.reshape(n, 1), batch, b_conv, W1, b1, W2, b2)


# speedup vs baseline: 19.0016x; 4.9690x over previous
import functools

import jax
import jax.numpy as jnp
from jax import lax
from jax.experimental import pallas as pl
from jax.experimental.pallas import tpu as pltpu
from jax.experimental.pallas import tpu_sc as plsc

N = 10000
NPAD = 10112
E = 320000
D = 128
NG = 64
NTILES = 32
EPT = E // NTILES
ECH = 79
EPTP = ECH * 128
RPT = NPAD // 16

_mesh = functools.partial(
    plsc.VectorSubcoreMesh, core_axis_name="c", subcore_axis_name="s")


HW = 8


def _hist_body(dst_hbm, ones_hbm, zeros_hbm, out_hbm, idx_v, ones_v, acc_sh, sem):
    c = lax.axis_index("c")
    s = lax.axis_index("s")
    wid = c * 16 + s
    base = s * RPT
    for k in range(RPT // 128):
        pltpu.sync_copy(zeros_hbm, acc_sh.at[pl.ds(base + k * 128, 128)])
    rem = RPT % 128
    if rem:
        pltpu.sync_copy(zeros_hbm.at[pl.ds(0, rem)],
                        acc_sh.at[pl.ds(base + (RPT // 128) * 128, rem)])
    pltpu.sync_copy(ones_hbm, ones_v)
    pltpu.sync_copy(dst_hbm.at[wid], idx_v)
    plsc.subcore_barrier()

    def step(j, carry):
        pltpu.sync_copy(ones_v, acc_sh.at[idx_v.at[j]], add=True)
        return carry

    lax.fori_loop(0, ECH, step, 0)
    plsc.subcore_barrier()
    pltpu.sync_copy(acc_sh.at[pl.ds(base, RPT)],
                    out_hbm.at[c].at[pl.ds(base, RPT)])


@functools.partial(
    pl.kernel,
    mesh=_mesh(),
    out_type=jax.ShapeDtypeStruct((2, NPAD, HW), jnp.float32),
    scratch_types=[
        pltpu.VMEM((ECH, 128), jnp.int32),
        pltpu.VMEM((128, HW), jnp.float32),
        pltpu.VMEM_SHARED((NPAD, HW), jnp.float32),
        pltpu.SemaphoreType.DMA,
    ],
)
def _sc_hist(dst_hbm, ones_hbm, zeros_hbm, out_hbm, idx_v, ones_v, acc_sh, sem):
    _hist_body(dst_hbm, ones_hbm, zeros_hbm, out_hbm, idx_v, ones_v, acc_sh, sem)


def _mid_body(x_ref, W_ref, hist_ref, g_ref, dinv_ref):
    deg = hist_ref[0, :, :1] + hist_ref[1, :, :1] + 1.0
    dinv = jax.lax.rsqrt(deg)
    dinv_ref[...] = dinv
    h = jax.lax.dot_general(x_ref[...], W_ref[...], (((1,), (0,)), ((), ())),
                            preferred_element_type=jnp.float32)
    g_ref[:N, :] = h * dinv[:N]
    g_ref[N:, :] = jnp.zeros((NPAD - N, D), jnp.float32)


def _tc_mid(x, W_conv, hist):
    return pl.pallas_call(
        _mid_body,
        out_shape=(jax.ShapeDtypeStruct((NPAD, D), jnp.float32),
                   jax.ShapeDtypeStruct((NPAD, 1), jnp.float32)),
    )(x, W_conv, hist)


def _scat_body(g_hbm, src_hbm, dst_hbm, zeros_hbm, out_hbm,
               src_v, dst_v, buf, acc_sh, sem):
    c = lax.axis_index("c")
    s = lax.axis_index("s")
    wid = c * 16 + s
    base = s * RPT
    for k in range(RPT // 128):
        pltpu.sync_copy(zeros_hbm, acc_sh.at[pl.ds(base + k * 128, 128)])
    rem = RPT % 128
    pltpu.sync_copy(zeros_hbm.at[pl.ds(0, rem)],
                    acc_sh.at[pl.ds(base + (RPT // 128) * 128, rem)])
    pltpu.sync_copy(src_hbm.at[wid], src_v)
    pltpu.sync_copy(dst_hbm.at[wid], dst_v)
    plsc.subcore_barrier()

    def step(j, carry):
        pltpu.async_copy(g_hbm.at[src_v.at[j]], buf, sem).wait()
        pltpu.sync_copy(buf, acc_sh.at[dst_v.at[j]], add=True)
        return carry

    lax.fori_loop(0, ECH, step, 0)
    plsc.subcore_barrier()
    pltpu.sync_copy(acc_sh.at[pl.ds(base, RPT)],
                    out_hbm.at[c].at[pl.ds(base, RPT)])


@functools.partial(
    pl.kernel,
    mesh=_mesh(),
    out_type=jax.ShapeDtypeStruct((2, NPAD, D), jnp.float32),
    scratch_types=[
        pltpu.VMEM((ECH, 128), jnp.int32),
        pltpu.VMEM((ECH, 128), jnp.int32),
        pltpu.VMEM((128, D), jnp.float32),
        pltpu.VMEM_SHARED((NPAD, D), jnp.float32),
        pltpu.SemaphoreType.DMA,
    ],
)
def _sc_scatter(g_hbm, src_hbm, dst_hbm, zeros_hbm, out_hbm,
                src_v, dst_v, buf, acc_sh, sem):
    _scat_body(g_hbm, src_hbm, dst_hbm, zeros_hbm, out_hbm,
               src_v, dst_v, buf, acc_sh, sem)


def _finish_body(parts_ref, g_ref, dinv_ref, batch_ref, b_conv_ref,
                 W1_ref, b1_ref, W2_ref, b2_ref, out_ref):
    acc = parts_ref[0, :N, :] + parts_ref[1, :N, :] + g_ref[:N, :]
    out_node = jnp.maximum(acc * dinv_ref[:N] + b_conv_ref[...], 0.0)
    gids = jax.lax.broadcasted_iota(jnp.int32, (N, NG), 1)
    P = jnp.where(batch_ref[...] == gids, 1.0, 0.0)
    sums = jax.lax.dot_general(P, out_node, (((0,), (0,)), ((), ())),
                               preferred_element_type=jnp.float32)
    cnts = jnp.sum(P, axis=0, keepdims=True)
    pooled = sums / jnp.maximum(cnts, 1.0).T
    z = jnp.maximum(
        jax.lax.dot_general(pooled, W1_ref[...], (((1,), (0,)), ((), ())),
                            preferred_element_type=jnp.float32) + b1_ref[...], 0.0)
    out_ref[...] = jax.lax.dot_general(z, W2_ref[...], (((1,), (0,)), ((), ())),
                                       preferred_element_type=jnp.float32) + b2_ref[...]


def _finish(parts, g, dinv, batch, b_conv, W1, b1, W2, b2):
    return pl.pallas_call(
        _finish_body,
        out_shape=jax.ShapeDtypeStruct((NG, 10), jnp.float32),
    )(parts, g, dinv, batch.reshape(N, 1), b_conv.reshape(1, -1),
      W1, b1.reshape(1, -1), W2, b2.reshape(1, -1))


def kernel(x, edge_index, batch, W_conv, b_conv, W1, b1, W2, b2):
    src = edge_index[0]
    dst = edge_index[1]
    pad = jnp.full((NTILES, EPTP - EPT), N, jnp.int32)
    src3 = jnp.concatenate([src.reshape(NTILES, EPT), pad], 1).reshape(
        NTILES, ECH, 128)
    dst3 = jnp.concatenate([dst.reshape(NTILES, EPT), pad], 1).reshape(
        NTILES, ECH, 128)
    ones_h = jnp.ones((128, HW), jnp.float32)
    zeros_h = jnp.zeros((128, HW), jnp.float32)
    hist = _sc_hist(dst3, ones_h, zeros_h)
    g_pad, dinv = _tc_mid(x, W_conv, hist)
    zeros2d = jnp.zeros((128, D), jnp.float32)
    parts = _sc_scatter(g_pad, src3, dst3, zeros2d)
    return _finish(parts, g_pad, dinv, batch, b_conv, W1, b1, W2, b2)
